# baseline (speedup 1.0000x reference)
import functools

import jax
import jax.numpy as jnp
from jax import lax
from jax.experimental import pallas as pl
from jax.experimental.pallas import tpu as pltpu
from jax.experimental.pallas import tpu_sc as plsc

N = 10000
K = 16
C = 64
R = 2
NPAD = 10240
F = 2 * C

NC = 2
NS = 16
NW = NC * NS
ROWS_PER_W = NPAD // NW
G = 8
QB = 512
RB = 1024


CT = 1024
BIGV = 3e38
BIGI = 2 ** 30


def _knn_body(xq_ref, xt_ref, bq_ref, bt_ref, idx_ref):
    i = pl.program_id(0)
    xq = xq_ref[...]
    qb = bq_ref[...]
    b_lo = jnp.min(qb)
    b_hi = jnp.max(qb)
    bt_full = bt_ref[...]
    start = jnp.sum((bt_full < b_lo).astype(jnp.int32))
    end = jnp.sum((bt_full <= b_hi).astype(jnp.int32))
    t0 = start // CT
    t1 = (end + CT - 1) // CT
    rowid = i * QB + lax.broadcasted_iota(jnp.int32, (QB, 1), 0)
    lane = lax.broadcasted_iota(jnp.int32, (QB, K), 1)
    vals0 = jnp.full((QB, K), BIGV, jnp.float32)
    ids0 = NPAD + lane

    sqq = (xq[:, 0:1] * xq[:, 0:1] + xq[:, 1:2] * xq[:, 1:2]
           + xq[:, 2:3] * xq[:, 2:3])

    def tile(ti, carry):
        vals, ids = carry
        off = pl.multiple_of(ti * CT, CT)
        xt = xt_ref[:, pl.ds(off, CT)]
        sq = (xt[0:1] * xt[0:1] + xt[1:2] * xt[1:2]
              + xt[2:3] * xt[2:3])
        dot = jnp.dot(xq, xt, preferred_element_type=jnp.float32)
        d = (sqq + sq) - 2.0 * dot
        colid = off + lax.broadcasted_iota(jnp.int32, (QB, CT), 1)
        mask = (qb != bt_ref[:, pl.ds(off, CT)]) | (colid == rowid)
        d = jnp.where(mask, 1e10, d)
        nv = vals0
        ni = ids0
        for t in range(K):
            m = jnp.minimum(jnp.min(d, 1, keepdims=True),
                            jnp.min(vals, 1, keepdims=True))
            am = jnp.minimum(
                jnp.min(jnp.where(d <= m, colid, BIGI), 1, keepdims=True),
                jnp.min(jnp.where(vals <= m, ids, BIGI), 1, keepdims=True))
            nv = jnp.where(lane == t, m, nv)
            ni = jnp.where(lane == t, am, ni)
            d = jnp.where(colid == am, BIGV, d)
            vals = jnp.where(ids == am, BIGV, vals)
        return nv, ni

    _, ids = lax.fori_loop(t0, t1, tile, (vals0, ids0))
    idx_ref[...] = ids


def _knn(xp, xt, bq, bt):
    return pl.pallas_call(
        _knn_body,
        grid=(NPAD // QB,),
        in_specs=[
            pl.BlockSpec((QB, 8), lambda i: (i, 0)),
            pl.BlockSpec((8, NPAD), lambda i: (0, 0)),
            pl.BlockSpec((QB, 1), lambda i: (i, 0)),
            pl.BlockSpec((1, NPAD), lambda i: (0, 0)),
        ],
        out_specs=pl.BlockSpec((QB, K), lambda i: (i, 0)),
        out_shape=jax.ShapeDtypeStruct((NPAD, K), jnp.int32),
    )(xp, xt, bq, bt)


def _lift_pq_body(x_ref, wl_ref, bl_ref, wa_ref, bp_ref, wb_ref,
                  h_ref, p_ref, q_ref):
    h = jnp.maximum(
        jnp.dot(x_ref[...], wl_ref[...], preferred_element_type=jnp.float32)
        + bl_ref[...], 0.0)
    h_ref[...] = h
    p_ref[...] = jnp.dot(h, wa_ref[...],
                         preferred_element_type=jnp.float32) + bp_ref[...]
    q_ref[...] = jnp.dot(h, wb_ref[...], preferred_element_type=jnp.float32)


def _lift_pq(xp, wl, bl, wa, bp, wb):
    return pl.pallas_call(
        _lift_pq_body,
        grid=(NPAD // RB,),
        in_specs=[
            pl.BlockSpec((RB, 8), lambda i: (i, 0)),
            pl.BlockSpec((8, C), lambda i: (0, 0)),
            pl.BlockSpec((1, C), lambda i: (0, 0)),
            pl.BlockSpec((C, F), lambda i: (0, 0)),
            pl.BlockSpec((1, F), lambda i: (0, 0)),
            pl.BlockSpec((C, F), lambda i: (0, 0)),
        ],
        out_specs=[
            pl.BlockSpec((RB, C), lambda i: (i, 0)),
            pl.BlockSpec((RB, F), lambda i: (i, 0)),
            pl.BlockSpec((RB, F), lambda i: (i, 0)),
        ],
        out_shape=[
            jax.ShapeDtypeStruct((NPAD, C), jnp.float32),
            jax.ShapeDtypeStruct((NPAD, F), jnp.float32),
            jax.ShapeDtypeStruct((NPAD, F), jnp.float32),
        ],
    )(xp, wl, bl, wa, bp, wb)


def _pq_body(t_ref, wa_ref, bp_ref, wb_ref, p_ref, q_ref):
    t = t_ref[...]
    p_ref[...] = jnp.dot(t, wa_ref[...],
                         preferred_element_type=jnp.float32) + bp_ref[...]
    q_ref[...] = jnp.dot(t, wb_ref[...], preferred_element_type=jnp.float32)


def _pq(t, wa, bp, wb):
    return pl.pallas_call(
        _pq_body,
        grid=(NPAD // RB,),
        in_specs=[
            pl.BlockSpec((RB, F), lambda i: (i, 0)),
            pl.BlockSpec((F, F), lambda i: (0, 0)),
            pl.BlockSpec((1, F), lambda i: (0, 0)),
            pl.BlockSpec((F, F), lambda i: (0, 0)),
        ],
        out_specs=[
            pl.BlockSpec((RB, F), lambda i: (i, 0)),
            pl.BlockSpec((RB, F), lambda i: (i, 0)),
        ],
        out_shape=[
            jax.ShapeDtypeStruct((NPAD, F), jnp.float32),
            jax.ShapeDtypeStruct((NPAD, F), jnp.float32),
        ],
    )(t, wa, bp, wb)


def _comb_pq_body(hp_ref, t_ref, wa_ref, bp_ref, wb_ref,
                  h_ref, p_ref, q_ref):
    t = t_ref[...]
    h = hp_ref[...] + 0.5 * (t[:, :C] + t[:, C:])
    h_ref[...] = h
    p_ref[...] = jnp.dot(h, wa_ref[...],
                         preferred_element_type=jnp.float32) + bp_ref[...]
    q_ref[...] = jnp.dot(h, wb_ref[...], preferred_element_type=jnp.float32)


def _comb_pq(hp, t, wa, bp, wb):
    return pl.pallas_call(
        _comb_pq_body,
        grid=(NPAD // RB,),
        in_specs=[
            pl.BlockSpec((RB, C), lambda i: (i, 0)),
            pl.BlockSpec((RB, F), lambda i: (i, 0)),
            pl.BlockSpec((C, F), lambda i: (0, 0)),
            pl.BlockSpec((1, F), lambda i: (0, 0)),
            pl.BlockSpec((C, F), lambda i: (0, 0)),
        ],
        out_specs=[
            pl.BlockSpec((RB, C), lambda i: (i, 0)),
            pl.BlockSpec((RB, F), lambda i: (i, 0)),
            pl.BlockSpec((RB, F), lambda i: (i, 0)),
        ],
        out_shape=[
            jax.ShapeDtypeStruct((NPAD, C), jnp.float32),
            jax.ShapeDtypeStruct((NPAD, F), jnp.float32),
            jax.ShapeDtypeStruct((NPAD, F), jnp.float32),
        ],
    )(hp, t, wa, bp, wb)


def _recon_body(u_ref, w1_ref, b1_ref, w2_ref, b2_ref, o_ref):
    t = jnp.maximum(
        jnp.dot(u_ref[...], w1_ref[...], preferred_element_type=jnp.float32)
        + b1_ref[...], 0.0)
    o_ref[...] = jnp.dot(t, w2_ref[...],
                         preferred_element_type=jnp.float32) + b2_ref[...]


def _recon(up, w1, b1, w2, b2):
    nrows = 2 * NPAD
    return pl.pallas_call(
        _recon_body,
        grid=(nrows // RB,),
        in_specs=[
            pl.BlockSpec((RB, C), lambda i: (i, 0)),
            pl.BlockSpec((C, C), lambda i: (0, 0)),
            pl.BlockSpec((1, C), lambda i: (0, 0)),
            pl.BlockSpec((C, F), lambda i: (0, 0)),
            pl.BlockSpec((1, F), lambda i: (0, 0)),
        ],
        out_specs=pl.BlockSpec((RB, F), lambda i: (i, 0)),
        out_shape=jax.ShapeDtypeStruct((nrows, F), jnp.float32),
    )(up, w1, b1, w2, b2)


NCH = ROWS_PER_W // G


def _gathermax_body(p_hbm, q_hbm, idx_hbm, out_hbm,
                    idx_v, p_v, o_v, r0, r1, s0, s1):
    wid = lax.axis_index("s") * NC + lax.axis_index("c")
    base = wid * ROWS_PER_W
    pltpu.sync_copy(idx_hbm.at[pl.ds(wid * NCH, NCH)], idx_v)
    pltpu.sync_copy(p_hbm.at[pl.ds(base, ROWS_PER_W)], p_v)
    rows = (r0, r1)
    sems = (s0, s1)

    def start_gather(cc, b):
        pltpu.async_copy(q_hbm.at[idx_v.at[cc]], rows[b], sems[b])

    def wait_gather(b):
        pltpu.make_async_copy(q_hbm.at[idx_v.at[0]], rows[b], sems[b]).wait()

    start_gather(0, 0)
    start_gather(1, 1)

    def pair(i, carry):
        for b in range(2):
            c = 2 * i + b
            wait_gather(b)

            def node(g, carry2):
                loc = c * G + g
                for j in range(F // 16):
                    sl = pl.ds(j * 16, 16)
                    m = rows[b][g * K, sl]
                    for kk in range(1, K):
                        m = jnp.maximum(m, rows[b][g * K + kk, sl])
                    o_v[loc, sl] = jnp.maximum(p_v[loc, sl] + m, 0.0)
                return carry2

            lax.fori_loop(0, G, node, 0)

            @pl.when(c + 2 < NCH)
            def _():
                start_gather(c + 2, b)
        return carry

    lax.fori_loop(0, NCH // 2, pair, 0)
    pltpu.sync_copy(o_v, out_hbm.at[pl.ds(base, ROWS_PER_W)])


@functools.cache
def _gathermax_built():
    return pl.kernel(
        _gathermax_body,
        out_type=jax.ShapeDtypeStruct((NPAD, F), jnp.float32),
        mesh=plsc.VectorSubcoreMesh(core_axis_name="c", subcore_axis_name="s",
                                    num_cores=NC, num_subcores=NS),
        scratch_types=[
            pltpu.VMEM((NCH, G * K), jnp.int32),
            pltpu.VMEM((ROWS_PER_W, F), jnp.float32),
            pltpu.VMEM((ROWS_PER_W, F), jnp.float32),
            pltpu.VMEM((G * K, F), jnp.float32),
            pltpu.VMEM((G * K, F), jnp.float32),
            pltpu.SemaphoreType.DMA,
            pltpu.SemaphoreType.DMA,
        ],
    )


def _gathermax(p, q, idxf):
    return _gathermax_built()(p, q, idxf.reshape(NPAD // G, G * K))


def kernel(x, batch, W_lift, b_lift, Wg, bg, W_sh, b_sh, W_r1, b_r1,
           W_r2, b_r2):
    f32 = jnp.float32
    xp = jnp.zeros((NPAD, 8), f32).at[:N, :3].set(x)
    bt = jnp.full((NPAD,), 127, jnp.int32).at[:N].set(batch.astype(jnp.int32))
    xt = xp.T
    idx = _knn(xp, xt, bt.reshape(NPAD, 1), bt.reshape(1, NPAD))
    idxf = idx.reshape(NPAD * K)

    def ab(i, br, j):
        w = Wg[i, br, j]
        return w[:C] - w[C:], w[C:], bg[i, br, j]

    def pack_first(i):
        a0, b0, c0 = ab(i, 0, 0)
        a1, b1, c1 = ab(i, 1, 0)
        return (jnp.concatenate([a0, a1], axis=1),
                jnp.concatenate([c0, c1]).reshape(1, F),
                jnp.concatenate([b0, b1], axis=1))

    def pack_second(i):
        a0, b0, c0 = ab(i, 0, 1)
        a1, b1, c1 = ab(i, 1, 1)
        wa = jnp.zeros((F, F), f32).at[:C, :C].set(a0).at[C:, C:].set(a1)
        wb = jnp.zeros((F, F), f32).at[:C, :C].set(b0).at[C:, C:].set(b1)
        return wa, jnp.concatenate([c0, c1]).reshape(1, F), wb

    wl = jnp.zeros((8, C), f32).at[:3].set(W_lift)
    wa0, bp0, wb0 = pack_first(0)
    h0, P, Q = _lift_pq(xp, wl, b_lift.reshape(1, C), wa0, bp0, wb0)
    T = _gathermax(P, Q, idxf)
    wa0b, bp0b, wb0b = pack_second(0)
    P, Q = _pq(T, wa0b, bp0b, wb0b)
    T = _gathermax(P, Q, idxf)

    wa1, bp1, wb1 = pack_first(1)
    h1, P, Q = _comb_pq(h0, T, wa1, bp1, wb1)
    T = _gathermax(P, Q, idxf)
    wa1b, bp1b, wb1b = pack_second(1)
    P, Q = _pq(T, wa1b, bp1b, wb1b)
    T = _gathermax(P, Q, idxf)

    a_sh = W_sh[:C] - W_sh[C:]
    h2, Psh, Qsh = _comb_pq(h1, T, a_sh, b_sh.reshape(1, F), W_sh[C:])
    agg = _gathermax(Psh, Qsh, idxf)

    up = agg.reshape(2 * NPAD, C)
    w2p = jnp.zeros((C, F), f32).at[:, :3].set(W_r2)
    b2p = jnp.zeros((1, F), f32).at[0, :3].set(b_r2)
    outp = _recon(up, W_r1, b_r1.reshape(1, C), w2p, b2p)
    return outp[:N * R, :3]


# speedup vs baseline: 1.0997x; 1.0997x over previous
import functools

import jax
import jax.numpy as jnp
from jax import lax
from jax.experimental import pallas as pl
from jax.experimental.pallas import tpu as pltpu
from jax.experimental.pallas import tpu_sc as plsc

N = 10000
K = 16
C = 64
R = 2
NPAD = 10240
F = 2 * C

NC = 2
NS = 16
NW = NC * NS
ROWS_PER_W = NPAD // NW
G = 8
QB = 256
RB = 1024


CT = 1024
BIGV = 3e38
BIGI = 2 ** 30


def _knn_body(xq_ref, xt_ref, bq_ref, bt_ref, idx_ref):
    i = pl.program_id(0)
    xq = xq_ref[...]
    qb = bq_ref[...]
    b_lo = jnp.min(qb)
    b_hi = jnp.max(qb)
    bt_full = bt_ref[...]
    start = jnp.sum((bt_full < b_lo).astype(jnp.int32))
    end = jnp.sum((bt_full <= b_hi).astype(jnp.int32))
    t0 = start // CT
    t1 = (end + CT - 1) // CT
    rowid = i * QB + lax.broadcasted_iota(jnp.int32, (QB, 1), 0)
    lane = lax.broadcasted_iota(jnp.int32, (QB, K), 1)
    vals0 = jnp.full((QB, K), BIGV, jnp.float32)
    ids0 = NPAD + lane

    sqq = (xq[:, 0:1] * xq[:, 0:1] + xq[:, 1:2] * xq[:, 1:2]
           + xq[:, 2:3] * xq[:, 2:3])

    def tile(ti, carry):
        vals, ids = carry
        off = pl.multiple_of(ti * CT, CT)
        xt = xt_ref[:, pl.ds(off, CT)]
        sq = (xt[0:1] * xt[0:1] + xt[1:2] * xt[1:2]
              + xt[2:3] * xt[2:3])
        dot = jnp.dot(xq, xt, preferred_element_type=jnp.float32)
        d = (sqq + sq) - 2.0 * dot
        colid = off + lax.broadcasted_iota(jnp.int32, (QB, CT), 1)
        mask = (qb != bt_ref[:, pl.ds(off, CT)]) | (colid == rowid)
        d = jnp.where(mask, 1e10, d)
        nv = vals0
        ni = ids0
        for t in range(K):
            m = jnp.minimum(jnp.min(d, 1, keepdims=True),
                            jnp.min(vals, 1, keepdims=True))
            am = jnp.minimum(
                jnp.min(jnp.where(d <= m, colid, BIGI), 1, keepdims=True),
                jnp.min(jnp.where(vals <= m, ids, BIGI), 1, keepdims=True))
            nv = jnp.where(lane == t, m, nv)
            ni = jnp.where(lane == t, am, ni)
            d = jnp.where(colid == am, BIGV, d)
            vals = jnp.where(ids == am, BIGV, vals)
        return nv, ni

    _, ids = lax.fori_loop(t0, t1, tile, (vals0, ids0))
    idx_ref[...] = ids


def _knn(xp, xt, bq, bt):
    return pl.pallas_call(
        _knn_body,
        grid=(NPAD // QB,),
        in_specs=[
            pl.BlockSpec((QB, 8), lambda i: (i, 0)),
            pl.BlockSpec((8, NPAD), lambda i: (0, 0)),
            pl.BlockSpec((QB, 1), lambda i: (i, 0)),
            pl.BlockSpec((1, NPAD), lambda i: (0, 0)),
        ],
        out_specs=pl.BlockSpec((QB, K), lambda i: (i, 0)),
        out_shape=jax.ShapeDtypeStruct((NPAD, K), jnp.int32),
    )(xp, xt, bq, bt)


def _lift_pq_body(x_ref, wl_ref, bl_ref, wa_ref, bp_ref, wb_ref,
                  h_ref, p_ref, q_ref):
    h = jnp.maximum(
        jnp.dot(x_ref[...], wl_ref[...], preferred_element_type=jnp.float32)
        + bl_ref[...], 0.0)
    h_ref[...] = h
    p_ref[...] = jnp.dot(h, wa_ref[...],
                         preferred_element_type=jnp.float32) + bp_ref[...]
    q_ref[...] = jnp.dot(h, wb_ref[...], preferred_element_type=jnp.float32)


def _lift_pq(xp, wl, bl, wa, bp, wb):
    return pl.pallas_call(
        _lift_pq_body,
        grid=(NPAD // RB,),
        in_specs=[
            pl.BlockSpec((RB, 8), lambda i: (i, 0)),
            pl.BlockSpec((8, C), lambda i: (0, 0)),
            pl.BlockSpec((1, C), lambda i: (0, 0)),
            pl.BlockSpec((C, F), lambda i: (0, 0)),
            pl.BlockSpec((1, F), lambda i: (0, 0)),
            pl.BlockSpec((C, F), lambda i: (0, 0)),
        ],
        out_specs=[
            pl.BlockSpec((RB, C), lambda i: (i, 0)),
            pl.BlockSpec((RB, F), lambda i: (i, 0)),
            pl.BlockSpec((RB, F), lambda i: (i, 0)),
        ],
        out_shape=[
            jax.ShapeDtypeStruct((NPAD, C), jnp.float32),
            jax.ShapeDtypeStruct((NPAD, F), jnp.float32),
            jax.ShapeDtypeStruct((NPAD, F), jnp.float32),
        ],
    )(xp, wl, bl, wa, bp, wb)


def _pq_body(t_ref, wa_ref, bp_ref, wb_ref, p_ref, q_ref):
    t = t_ref[...]
    p_ref[...] = jnp.dot(t, wa_ref[...],
                         preferred_element_type=jnp.float32) + bp_ref[...]
    q_ref[...] = jnp.dot(t, wb_ref[...], preferred_element_type=jnp.float32)


def _pq(t, wa, bp, wb):
    return pl.pallas_call(
        _pq_body,
        grid=(NPAD // RB,),
        in_specs=[
            pl.BlockSpec((RB, F), lambda i: (i, 0)),
            pl.BlockSpec((F, F), lambda i: (0, 0)),
            pl.BlockSpec((1, F), lambda i: (0, 0)),
            pl.BlockSpec((F, F), lambda i: (0, 0)),
        ],
        out_specs=[
            pl.BlockSpec((RB, F), lambda i: (i, 0)),
            pl.BlockSpec((RB, F), lambda i: (i, 0)),
        ],
        out_shape=[
            jax.ShapeDtypeStruct((NPAD, F), jnp.float32),
            jax.ShapeDtypeStruct((NPAD, F), jnp.float32),
        ],
    )(t, wa, bp, wb)


def _comb_pq_body(hp_ref, t_ref, wa_ref, bp_ref, wb_ref,
                  h_ref, p_ref, q_ref):
    t = t_ref[...]
    h = hp_ref[...] + 0.5 * (t[:, :C] + t[:, C:])
    h_ref[...] = h
    p_ref[...] = jnp.dot(h, wa_ref[...],
                         preferred_element_type=jnp.float32) + bp_ref[...]
    q_ref[...] = jnp.dot(h, wb_ref[...], preferred_element_type=jnp.float32)


def _comb_pq(hp, t, wa, bp, wb):
    return pl.pallas_call(
        _comb_pq_body,
        grid=(NPAD // RB,),
        in_specs=[
            pl.BlockSpec((RB, C), lambda i: (i, 0)),
            pl.BlockSpec((RB, F), lambda i: (i, 0)),
            pl.BlockSpec((C, F), lambda i: (0, 0)),
            pl.BlockSpec((1, F), lambda i: (0, 0)),
            pl.BlockSpec((C, F), lambda i: (0, 0)),
        ],
        out_specs=[
            pl.BlockSpec((RB, C), lambda i: (i, 0)),
            pl.BlockSpec((RB, F), lambda i: (i, 0)),
            pl.BlockSpec((RB, F), lambda i: (i, 0)),
        ],
        out_shape=[
            jax.ShapeDtypeStruct((NPAD, C), jnp.float32),
            jax.ShapeDtypeStruct((NPAD, F), jnp.float32),
            jax.ShapeDtypeStruct((NPAD, F), jnp.float32),
        ],
    )(hp, t, wa, bp, wb)


def _recon_body(u_ref, w1_ref, b1_ref, w2_ref, b2_ref, o_ref):
    t = jnp.maximum(
        jnp.dot(u_ref[...], w1_ref[...], preferred_element_type=jnp.float32)
        + b1_ref[...], 0.0)
    o_ref[...] = jnp.dot(t, w2_ref[...],
                         preferred_element_type=jnp.float32) + b2_ref[...]


def _recon(up, w1, b1, w2, b2):
    nrows = 2 * NPAD
    return pl.pallas_call(
        _recon_body,
        grid=(nrows // RB,),
        in_specs=[
            pl.BlockSpec((RB, C), lambda i: (i, 0)),
            pl.BlockSpec((C, C), lambda i: (0, 0)),
            pl.BlockSpec((1, C), lambda i: (0, 0)),
            pl.BlockSpec((C, F), lambda i: (0, 0)),
            pl.BlockSpec((1, F), lambda i: (0, 0)),
        ],
        out_specs=pl.BlockSpec((RB, F), lambda i: (i, 0)),
        out_shape=jax.ShapeDtypeStruct((nrows, F), jnp.float32),
    )(up, w1, b1, w2, b2)


NCH = ROWS_PER_W // G


def _gathermax_body(p_hbm, q_hbm, idx_hbm, out_hbm,
                    idx_v, p_v, o_v, r0, r1, s0, s1):
    wid = lax.axis_index("s") * NC + lax.axis_index("c")
    base = wid * ROWS_PER_W
    pltpu.sync_copy(idx_hbm.at[pl.ds(wid * NCH, NCH)], idx_v)
    pltpu.sync_copy(p_hbm.at[pl.ds(base, ROWS_PER_W)], p_v)
    rows = (r0, r1)
    sems = (s0, s1)

    def start_gather(cc, b):
        pltpu.async_copy(q_hbm.at[idx_v.at[cc]], rows[b], sems[b])

    def wait_gather(b):
        pltpu.make_async_copy(q_hbm.at[idx_v.at[0]], rows[b], sems[b]).wait()

    start_gather(0, 0)
    start_gather(1, 1)

    def pair(i, carry):
        for b in range(2):
            c = 2 * i + b
            wait_gather(b)

            def node(g, carry2):
                loc = c * G + g
                for j in range(F // 16):
                    sl = pl.ds(j * 16, 16)
                    m = rows[b][g * K, sl]
                    for kk in range(1, K):
                        m = jnp.maximum(m, rows[b][g * K + kk, sl])
                    o_v[loc, sl] = jnp.maximum(p_v[loc, sl] + m, 0.0)
                return carry2

            lax.fori_loop(0, G, node, 0)

            @pl.when(c + 2 < NCH)
            def _():
                start_gather(c + 2, b)
        return carry

    lax.fori_loop(0, NCH // 2, pair, 0)
    pltpu.sync_copy(o_v, out_hbm.at[pl.ds(base, ROWS_PER_W)])


@functools.cache
def _gathermax_built():
    return pl.kernel(
        _gathermax_body,
        out_type=jax.ShapeDtypeStruct((NPAD, F), jnp.float32),
        mesh=plsc.VectorSubcoreMesh(core_axis_name="c", subcore_axis_name="s",
                                    num_cores=NC, num_subcores=NS),
        scratch_types=[
            pltpu.VMEM((NCH, G * K), jnp.int32),
            pltpu.VMEM((ROWS_PER_W, F), jnp.float32),
            pltpu.VMEM((ROWS_PER_W, F), jnp.float32),
            pltpu.VMEM((G * K, F), jnp.float32),
            pltpu.VMEM((G * K, F), jnp.float32),
            pltpu.SemaphoreType.DMA,
            pltpu.SemaphoreType.DMA,
        ],
    )


def _gathermax(p, q, idxf):
    return _gathermax_built()(p, q, idxf.reshape(NPAD // G, G * K))


def kernel(x, batch, W_lift, b_lift, Wg, bg, W_sh, b_sh, W_r1, b_r1,
           W_r2, b_r2):
    f32 = jnp.float32
    xp = jnp.zeros((NPAD, 8), f32).at[:N, :3].set(x)
    bt = jnp.full((NPAD,), 127, jnp.int32).at[:N].set(batch.astype(jnp.int32))
    xt = xp.T
    idx = _knn(xp, xt, bt.reshape(NPAD, 1), bt.reshape(1, NPAD))
    idxf = idx.reshape(NPAD * K)

    def ab(i, br, j):
        w = Wg[i, br, j]
        return w[:C] - w[C:], w[C:], bg[i, br, j]

    def pack_first(i):
        a0, b0, c0 = ab(i, 0, 0)
        a1, b1, c1 = ab(i, 1, 0)
        return (jnp.concatenate([a0, a1], axis=1),
                jnp.concatenate([c0, c1]).reshape(1, F),
                jnp.concatenate([b0, b1], axis=1))

    def pack_second(i):
        a0, b0, c0 = ab(i, 0, 1)
        a1, b1, c1 = ab(i, 1, 1)
        wa = jnp.zeros((F, F), f32).at[:C, :C].set(a0).at[C:, C:].set(a1)
        wb = jnp.zeros((F, F), f32).at[:C, :C].set(b0).at[C:, C:].set(b1)
        return wa, jnp.concatenate([c0, c1]).reshape(1, F), wb

    wl = jnp.zeros((8, C), f32).at[:3].set(W_lift)
    wa0, bp0, wb0 = pack_first(0)
    h0, P, Q = _lift_pq(xp, wl, b_lift.reshape(1, C), wa0, bp0, wb0)
    T = _gathermax(P, Q, idxf)
    wa0b, bp0b, wb0b = pack_second(0)
    P, Q = _pq(T, wa0b, bp0b, wb0b)
    T = _gathermax(P, Q, idxf)

    wa1, bp1, wb1 = pack_first(1)
    h1, P, Q = _comb_pq(h0, T, wa1, bp1, wb1)
    T = _gathermax(P, Q, idxf)
    wa1b, bp1b, wb1b = pack_second(1)
    P, Q = _pq(T, wa1b, bp1b, wb1b)
    T = _gathermax(P, Q, idxf)

    a_sh = W_sh[:C] - W_sh[C:]
    h2, Psh, Qsh = _comb_pq(h1, T, a_sh, b_sh.reshape(1, F), W_sh[C:])
    agg = _gathermax(Psh, Qsh, idxf)

    up = agg.reshape(2 * NPAD, C)
    w2p = jnp.zeros((C, F), f32).at[:, :3].set(W_r2)
    b2p = jnp.zeros((1, F), f32).at[0, :3].set(b_r2)
    outp = _recon(up, W_r1, b_r1.reshape(1, C), w2p, b2p)
    return outp[:N * R, :3]
